# baseline (speedup 1.0000x reference)
import jax
import jax.numpy as jnp
from jax import lax
from jax.experimental import pallas as pl
from jax.experimental.pallas import tpu as pltpu

B = 1024
S = 50
IN_CH = 128
OUT_CH = 256
D = 128
N_OUT = 50001
NNZ = 64

BT = 128
CB = 2048

_T_RHS = (((1,), (1,)), ((), ()))
_PARALLEL = pltpu.CompilerParams(dimension_semantics=("parallel",))


def _encode_body(x_ref, cw_ref, lw_ref, cb_ref, lb_ref, gb_ref,
                 phrase_ref, base_ref):
    cw = cw_ref[:]
    m = jnp.full((BT, OUT_CH), -jnp.inf, dtype=jnp.float32)
    for s in range(S):
        xs = x_ref[:, s * IN_CH:(s + 1) * IN_CH]
        z = lax.dot_general(xs, cw, _T_RHS,
                            preferred_element_type=jnp.float32)
        m = jnp.maximum(m, z)
    h1 = jnp.maximum(m + cb_ref[:], 0.0)
    h2 = lax.dot_general(h1, lw_ref[:], _T_RHS,
                         preferred_element_type=jnp.float32) + lb_ref[:]
    h2 = jnp.maximum(h2, 0.0)
    norm = jnp.maximum(
        jnp.sqrt(jnp.sum(h2 * h2, axis=1, keepdims=True)), 1e-12)
    phrase = h2 / norm
    phrase_ref[:] = phrase
    base_ref[:] = jnp.sum(phrase * gb_ref[:], axis=1, keepdims=True)


def _encode(data2, cw, lw, cb, lb, gb):
    return pl.pallas_call(
        _encode_body,
        grid=(B // BT,),
        in_specs=[
            pl.BlockSpec((BT, S * IN_CH), lambda i: (i, 0)),
            pl.BlockSpec((OUT_CH, IN_CH), lambda i: (0, 0)),
            pl.BlockSpec((D, OUT_CH), lambda i: (0, 0)),
            pl.BlockSpec((1, OUT_CH), lambda i: (0, 0)),
            pl.BlockSpec((1, D), lambda i: (0, 0)),
            pl.BlockSpec((1, D), lambda i: (0, 0)),
        ],
        out_specs=[
            pl.BlockSpec((BT, D), lambda i: (i, 0)),
            pl.BlockSpec((BT, 1), lambda i: (i, 0)),
        ],
        out_shape=[
            jax.ShapeDtypeStruct((B, D), jnp.float32),
            jax.ShapeDtypeStruct((B, 1), jnp.float32),
        ],
        compiler_params=_PARALLEL,
    )(data2, cw, lw, cb, lb, gb)


def _gather_corr_body(idx1_ref, phrase_ref, v_ref, h0_ref, corr_ref,
                      gbuf, gsem):
    gcopies = [
        pltpu.make_async_copy(h0_ref.at[pl.ds(idx1_ref[k], 1), :],
                              gbuf.at[k, :, :], gsem)
        for k in range(NNZ)
    ]
    for c in gcopies:
        c.start()
    for c in gcopies:
        c.wait()
    g = gbuf[:, 0, :]
    corr_ref[:] = lax.dot_general(phrase_ref[:], g, _T_RHS,
                                  preferred_element_type=jnp.float32) * v_ref[:]


def _gather_corr(idx1, phrase, vals, h0):
    grid_spec = pltpu.PrefetchScalarGridSpec(
        num_scalar_prefetch=1,
        grid=(1,),
        in_specs=[
            pl.BlockSpec((B, D), lambda i, idx: (0, 0)),
            pl.BlockSpec((1, NNZ), lambda i, idx: (0, 0)),
            pl.BlockSpec(memory_space=pl.ANY),
        ],
        out_specs=pl.BlockSpec((B, NNZ), lambda i, idx: (0, 0)),
        scratch_shapes=[
            pltpu.VMEM((NNZ, 1, D), jnp.float32),
            pltpu.SemaphoreType.DMA,
        ],
    )
    return pl.pallas_call(
        _gather_corr_body,
        grid_spec=grid_spec,
        out_shape=jax.ShapeDtypeStruct((B, NNZ), jnp.float32),
    )(idx1, phrase, vals, h0)


def _fill_body(corr_ref, base_ref, idx0_ref, out_ref):
    j = pl.program_id(0)
    col0 = j * CB
    idx0 = idx0_ref[:]
    base = base_ref[:]
    hit = jnp.any((idx0 >= col0) & (idx0 < col0 + CB))

    @pl.when(hit)
    def _():
        cols = lax.broadcasted_iota(jnp.int32, (NNZ, CB), 1) + col0
        onehot = (cols == idx0).astype(jnp.float32)
        out_ref[:] = jnp.dot(corr_ref[:], onehot,
                             preferred_element_type=jnp.float32) + base

    @pl.when(jnp.logical_not(hit))
    def _():
        out_ref[:] = jnp.broadcast_to(base, (B, CB))


def _fill(corr, base, idx0):
    return pl.pallas_call(
        _fill_body,
        grid=(pl.cdiv(N_OUT, CB),),
        in_specs=[
            pl.BlockSpec((B, NNZ), lambda j: (0, 0)),
            pl.BlockSpec((B, 1), lambda j: (0, 0)),
            pl.BlockSpec((NNZ, 1), lambda j: (0, 0)),
        ],
        out_specs=pl.BlockSpec((B, CB), lambda j: (0, j)),
        out_shape=jax.ShapeDtypeStruct((B, N_OUT), jnp.float32),
        compiler_params=_PARALLEL,
    )(corr, base, idx0)


def kernel(data, seq_len, conv_w, conv_b, lin_w, lin_b, H0, gcn_bias, indices, values):
    del seq_len
    phrase, base = _encode(
        data.reshape(B, S * IN_CH),
        conv_w, lin_w,
        conv_b.reshape(1, OUT_CH), lin_b.reshape(1, D),
        gcn_bias.reshape(1, D),
    )
    corr = _gather_corr(indices[1], phrase, values.reshape(1, NNZ), H0)
    return _fanout_probe(base)


NSEM = 8
REM = N_OUT - (N_OUT // CB) * CB
NBLK_FULL = N_OUT // CB


def _fanout_probe_body(base_ref, out_ref, buf, tailbuf, sems):
    buf[:] = jnp.broadcast_to(base_ref[:], (B, CB))
    tailbuf[:] = jnp.broadcast_to(base_ref[:], (B, REM))
    cps = [pltpu.make_async_copy(buf, out_ref.at[:, pl.ds(j * CB, CB)],
                                 sems.at[j % NSEM])
           for j in range(NBLK_FULL)]
    cps.append(pltpu.make_async_copy(
        tailbuf, out_ref.at[:, pl.ds(NBLK_FULL * CB, REM)],
        sems.at[NBLK_FULL % NSEM]))
    for c in cps:
        c.start()
    for c in cps:
        c.wait()


def _fanout_probe(base):
    return pl.pallas_call(
        _fanout_probe_body,
        grid=(1,),
        in_specs=[pl.BlockSpec((B, 1), lambda i: (0, 0))],
        out_specs=pl.BlockSpec(memory_space=pl.ANY),
        out_shape=jax.ShapeDtypeStruct((B, N_OUT), jnp.float32),
        scratch_shapes=[
            pltpu.VMEM((B, CB), jnp.float32),
            pltpu.VMEM((B, REM), jnp.float32),
            pltpu.SemaphoreType.DMA((NSEM,)),
        ],
    )(base)


BR = 32


def _rowfill_probe_body(base_ref, out_ref):
    out_ref[:] = jnp.broadcast_to(base_ref[:], (BR, N_OUT))


def _rowfill_probe(base):
    return pl.pallas_call(
        _rowfill_probe_body,
        grid=(B // BR,),
        in_specs=[pl.BlockSpec((BR, 1), lambda i: (i, 0))],
        out_specs=pl.BlockSpec((BR, N_OUT), lambda i: (i, 0)),
        out_shape=jax.ShapeDtypeStruct((B, N_OUT), jnp.float32),
        compiler_params=_PARALLEL,
    )(base)


# speedup vs baseline: 3.4793x; 3.4793x over previous
import jax
import jax.numpy as jnp
from jax import lax
from jax.experimental import pallas as pl
from jax.experimental.pallas import tpu as pltpu

B = 1024
S = 50
IN_CH = 128
OUT_CH = 256
D = 128
N_OUT = 50001
NNZ = 64

BT = 128
CBT = 2048

_T_RHS = (((1,), (1,)), ((), ()))
_PARALLEL = pltpu.CompilerParams(dimension_semantics=("parallel",))


def _encode_body(x_ref, cw_ref, lw_ref, cb_ref, lb_ref, phrase_ref):
    cw = cw_ref[:]
    m = jnp.full((BT, OUT_CH), -jnp.inf, dtype=jnp.float32)
    for s in range(S):
        xs = x_ref[s]
        z = lax.dot_general(xs, cw, _T_RHS,
                            preferred_element_type=jnp.float32)
        m = jnp.maximum(m, z)
    h1 = jnp.maximum(m + cb_ref[:], 0.0)
    h2 = lax.dot_general(h1, lw_ref[:], _T_RHS,
                         preferred_element_type=jnp.float32) + lb_ref[:]
    h2 = jnp.maximum(h2, 0.0)
    norm = jnp.maximum(
        jnp.sqrt(jnp.sum(h2 * h2, axis=1, keepdims=True)), 1e-12)
    phrase_ref[:] = h2 / norm


def _encode(dataT, cw, lw, cb, lb):
    return pl.pallas_call(
        _encode_body,
        grid=(B // BT,),
        in_specs=[
            pl.BlockSpec((S, BT, IN_CH), lambda i: (0, i, 0)),
            pl.BlockSpec((OUT_CH, IN_CH), lambda i: (0, 0)),
            pl.BlockSpec((D, OUT_CH), lambda i: (0, 0)),
            pl.BlockSpec((1, OUT_CH), lambda i: (0, 0)),
            pl.BlockSpec((1, D), lambda i: (0, 0)),
        ],
        out_specs=pl.BlockSpec((BT, D), lambda i: (i, 0)),
        out_shape=jax.ShapeDtypeStruct((B, D), jnp.float32),
        compiler_params=_PARALLEL,
    )(dataT, cw, lw, cb, lb)


def _gather_corr_body(idx1_ref, phrase_ref, v_ref, gb_ref, h0_ref, corrb_ref,
                      gbuf, gsem):
    gcopies = [
        pltpu.make_async_copy(h0_ref.at[pl.ds(idx1_ref[k], 1), :],
                              gbuf.at[k, :, :], gsem)
        for k in range(NNZ)
    ]
    for c in gcopies:
        c.start()
    for c in gcopies:
        c.wait()
    gs = gbuf[:, 0, :] * v_ref[:]
    gx = jnp.concatenate([gs, gb_ref[:]], axis=0)
    corrb_ref[:] = lax.dot_general(gx, phrase_ref[:], _T_RHS,
                                   preferred_element_type=jnp.float32)


def _gather_corr(idx1, phrase, v_col, gb, h0):
    grid_spec = pltpu.PrefetchScalarGridSpec(
        num_scalar_prefetch=1,
        grid=(1,),
        in_specs=[
            pl.BlockSpec((B, D), lambda i, idx: (0, 0)),
            pl.BlockSpec((NNZ, 1), lambda i, idx: (0, 0)),
            pl.BlockSpec((1, D), lambda i, idx: (0, 0)),
            pl.BlockSpec(memory_space=pl.ANY),
        ],
        out_specs=pl.BlockSpec((NNZ + 1, B), lambda i, idx: (0, 0)),
        scratch_shapes=[
            pltpu.VMEM((NNZ, 1, D), jnp.float32),
            pltpu.SemaphoreType.DMA,
        ],
    )
    return pl.pallas_call(
        _gather_corr_body,
        grid_spec=grid_spec,
        out_shape=jax.ShapeDtypeStruct((NNZ + 1, B), jnp.float32),
    )(idx1, phrase, v_col, gb, h0)


def _fill_body(corrb_ref, idx0x_ref, out_ref):
    j = pl.program_id(0)
    row0 = j * CBT
    idv = idx0x_ref[:]
    hit = jnp.any((idv >= row0) & (idv < row0 + CBT))

    @pl.when(hit)
    def _():
        rows = lax.broadcasted_iota(jnp.int32, (CBT, NNZ + 1), 0) + row0
        lane = lax.broadcasted_iota(jnp.int32, (CBT, NNZ + 1), 1)
        a = jnp.where((rows == idv) | (lane == NNZ), 1.0, 0.0)
        out_ref[:] = jnp.dot(a, corrb_ref[:],
                             preferred_element_type=jnp.float32)

    @pl.when(jnp.logical_not(hit))
    def _():
        out_ref[:] = jnp.broadcast_to(corrb_ref[NNZ:NNZ + 1, :], (CBT, B))


def _fill(corrb, idx0x):
    return pl.pallas_call(
        _fill_body,
        grid=(pl.cdiv(N_OUT, CBT),),
        in_specs=[
            pl.BlockSpec((NNZ + 1, B), lambda j: (0, 0)),
            pl.BlockSpec((1, NNZ + 1), lambda j: (0, 0)),
        ],
        out_specs=pl.BlockSpec((CBT, B), lambda j: (j, 0)),
        out_shape=jax.ShapeDtypeStruct((N_OUT, B), jnp.float32),
        compiler_params=_PARALLEL,
    )(corrb, idx0x)


def kernel(data, seq_len, conv_w, conv_b, lin_w, lin_b, H0, gcn_bias, indices, values):
    del seq_len
    phrase = _encode(
        data.transpose(1, 0, 2),
        conv_w, lin_w,
        conv_b.reshape(1, OUT_CH), lin_b.reshape(1, D),
    )
    corrb = _gather_corr(indices[1], phrase, values.reshape(NNZ, 1),
                         gcn_bias.reshape(1, D), H0)
    idx0x = jnp.concatenate(
        [indices[0], jnp.full((1,), -1, jnp.int32)]).reshape(1, NNZ + 1)
    return _fill(corrb, idx0x).T
